# baseline (speedup 1.0000x reference)
import jax
import jax.numpy as jnp
from jax import lax
from jax.experimental import pallas as pl
from jax.experimental.pallas import tpu as pltpu


def _body(vals_ref, x_ref, o_ref):
    xv = x_ref[...]
    v0 = vals_ref[0]
    v1 = vals_ref[1]
    v2 = vals_ref[2]
    s = lax.broadcasted_iota(jnp.int32, (4, 1), 0)
    zero = jnp.zeros((4, 1), jnp.float32)
    a = jnp.where(s == 0, v0, zero) + jnp.where(s == 1, v1, zero)
    b = jnp.where(s == 2, v2, zero)
    xs = pltpu.roll(xv, 3, 0)
    o_ref[...] = xv * a + xs * b


@jax.jit
def kernel(x, values):
    out_t = pl.pallas_call(
        _body,
        grid=(4,),
        out_shape=jax.ShapeDtypeStruct((4, 16384), jnp.float32),
        in_specs=[
            pl.BlockSpec(memory_space=pltpu.SMEM),
            pl.BlockSpec((4, 4096), lambda i: (0, i)),
        ],
        out_specs=pl.BlockSpec((4, 4096), lambda i: (0, i)),
    )(values, x.T)
    return out_t.T


# speedup vs baseline: 1.7963x; 1.7963x over previous
import jax
import jax.numpy as jnp
from jax import lax
from jax.experimental import pallas as pl
from jax.experimental.pallas import tpu as pltpu

_N = 16384


def _body(vals_ref, x_ref, o_ref):
    v0 = vals_ref[0]
    v1 = vals_ref[1]
    v2 = vals_ref[2]
    s = lax.broadcasted_iota(jnp.int32, (2, 1), 0)
    a = jnp.where(s == 0, v0, v1)
    o_ref[0:2, :] = x_ref[0:2, :] * a
    o_ref[2:3, :] = x_ref[3:4, :] * v2
    o_ref[3:4, :] = jnp.zeros((1, _N), jnp.float32)


@jax.jit
def kernel(x, values):
    out_t = pl.pallas_call(
        _body,
        out_shape=jax.ShapeDtypeStruct((4, _N), jnp.float32),
        in_specs=[
            pl.BlockSpec(memory_space=pltpu.SMEM),
            pl.BlockSpec(memory_space=pltpu.VMEM),
        ],
        out_specs=pl.BlockSpec(memory_space=pltpu.VMEM),
    )(values, x.T)
    return out_t.T
